# baseline (speedup 1.0000x reference)
import functools

import jax
import jax.numpy as jnp
from jax import lax
from jax.experimental import pallas as pl
from jax.experimental.pallas import tpu as pltpu
from jax.experimental.pallas import tpu_sc as plsc

SEQ = 200
BATCH = 16384
VOCAB = 1000000
NC = 2
NS = 16
NW = NC * NS
BPW = BATCH // NW
NCHUNK = BPW // 128
LANES = 16


NIBUF = 4
NVBUF = 2


def _lr_body(inp_hbm, lut_hbm, bias_hbm, out_hbm,
             table_sh, idx_v, vals_v, acc_v, bias_sm,
             isem0, isem1, isem2, isem3, gsem0, gsem1):
    cid = lax.axis_index("c")
    sid = lax.axis_index("s")
    wid = cid * NS + sid
    isems = (isem0, isem1, isem2, isem3)
    gsems = (gsem0, gsem1)

    @pl.when(sid == 0)
    def _():
        pltpu.sync_copy(lut_hbm, table_sh)

    bias_sm[...] = jnp.zeros((LANES,), jnp.float32)
    pltpu.sync_copy(bias_hbm, bias_sm.at[pl.ds(0, 1)])
    plsc.subcore_barrier()

    zero_idx = jnp.zeros((LANES,), jnp.int32)
    bvec = bias_sm[...].at[zero_idx].get(mode="promise_in_bounds")
    for i in range(BPW // LANES):
        acc_v[pl.ds(i * LANES, LANES)] = bvec

    def idx_desc(s, islot):
        return pltpu.make_async_copy(
            inp_hbm.at[s, pl.ds(wid * BPW, BPW)], idx_v.at[islot],
            isems[islot])

    def gather_descs(s, vslot, islot):
        del s
        return [
            pltpu.make_async_copy(
                table_sh.at[idx_v.at[islot, pl.ds(j * 128, 128)]],
                vals_v.at[vslot, j], gsems[vslot])
            for j in range(NCHUNK)
        ]

    def accum(vslot):
        for j in range(NCHUNK):
            for i in range(128 // LANES):
                v = vals_v[vslot, j, pl.ds(i * LANES, LANES)]
                plsc.addupdate(acc_v.at[pl.ds(j * 128 + i * LANES, LANES)], v)

    idx_desc(0, 0).start()
    idx_desc(1, 1).start()
    idx_desc(0, 0).wait()
    for d in gather_descs(0, 0, 0):
        d.start()

    def body(t, carry):
        s0 = t * NIBUF
        for k in range(NIBUF):
            s = s0 + k
            idx_desc(s + 2, (k + 2) % NIBUF).start()
            idx_desc(s + 1, (k + 1) % NIBUF).wait()
            for d in gather_descs(s + 1, (k + 1) % NVBUF, (k + 1) % NIBUF):
                d.start()
            for d in gather_descs(s, k % NVBUF, k % NIBUF):
                d.wait()
            accum(k % NVBUF)
        return carry

    lax.fori_loop(0, (SEQ - 4) // NIBUF, body, 0)

    for k in range(NIBUF):
        s = SEQ - 4 + k
        if k + 2 < NIBUF:
            idx_desc(s + 2, (k + 2) % NIBUF).start()
        if k + 1 < NIBUF:
            idx_desc(s + 1, (k + 1) % NIBUF).wait()
            for d in gather_descs(s + 1, (k + 1) % NVBUF, (k + 1) % NIBUF):
                d.start()
        for d in gather_descs(s, k % NVBUF, k % NIBUF):
            d.wait()
        accum(k % NVBUF)

    pltpu.sync_copy(acc_v, out_hbm.at[pl.ds(wid * BPW, BPW)])


@jax.jit
def kernel(input, lut, bias):
    inp = input
    lut_flat = jnp.pad(lut.reshape(VOCAB), (0, 448))
    run = pl.kernel(
        _lr_body,
        out_type=jax.ShapeDtypeStruct((BATCH,), jnp.float32),
        mesh=plsc.VectorSubcoreMesh(core_axis_name="c", subcore_axis_name="s"),
        scratch_types=[
            pltpu.VMEM_SHARED((VOCAB + 448,), jnp.float32),
            pltpu.VMEM((NIBUF, BPW), jnp.int32),
            pltpu.VMEM((NVBUF, NCHUNK, 128), jnp.float32),
            pltpu.VMEM((BPW,), jnp.float32),
            pltpu.VMEM((LANES,), jnp.float32),
            pltpu.SemaphoreType.DMA,
            pltpu.SemaphoreType.DMA,
            pltpu.SemaphoreType.DMA,
            pltpu.SemaphoreType.DMA,
            pltpu.SemaphoreType.DMA,
            pltpu.SemaphoreType.DMA,
        ],
    )
    return run(inp, lut_flat, bias)


# speedup vs baseline: 1.4209x; 1.4209x over previous
import functools

import jax
import jax.numpy as jnp
from jax import lax
from jax.experimental import pallas as pl
from jax.experimental.pallas import tpu as pltpu
from jax.experimental.pallas import tpu_sc as plsc

SEQ = 200
BATCH = 16384
VOCAB = 1000000
NC = 2
NS = 16
NW = NC * NS
BPW = BATCH // NW
NCHUNK = BPW // 128
LANES = 16


NIBUF = 4
NVBUF = 2


def _lr_body(inp_hbm, lut_hbm, bias_hbm, out_hbm,
             table_sh, idx_v, vals_v, acc_v, bias_sm,
             isem0, isem1, isem2, isem3, gsem0, gsem1):
    cid = lax.axis_index("c")
    sid = lax.axis_index("s")
    wid = cid * NS + sid
    isems = (isem0, isem1, isem2, isem3)
    gsems = (gsem0, gsem1)

    @pl.when(sid == 0)
    def _():
        pltpu.sync_copy(lut_hbm.at[0], table_sh)

    bias_sm[...] = jnp.zeros((LANES,), jnp.float32)
    pltpu.sync_copy(bias_hbm, bias_sm.at[pl.ds(0, 1)])
    plsc.subcore_barrier()

    zero_idx = jnp.zeros((LANES,), jnp.int32)
    bvec = bias_sm[...].at[zero_idx].get(mode="promise_in_bounds")
    for i in range(BPW // LANES):
        acc_v[pl.ds(i * LANES, LANES)] = bvec

    def idx_desc(s, islot):
        return pltpu.make_async_copy(
            inp_hbm.at[s, pl.ds(wid * BPW, BPW)], idx_v.at[islot],
            isems[islot])

    def gather_descs(s, vslot, islot):
        del s
        return [
            pltpu.make_async_copy(
                table_sh.at[idx_v.at[islot, pl.ds(j * 128, 128)]],
                vals_v.at[vslot, j], gsems[vslot])
            for j in range(NCHUNK)
        ]

    def accum(vslot):
        for j in range(NCHUNK):
            for i in range(128 // LANES):
                v = vals_v[vslot, j, pl.ds(i * LANES, LANES)]
                plsc.addupdate(acc_v.at[pl.ds(j * 128 + i * LANES, LANES)], v)

    idx_desc(0, 0).start()
    idx_desc(1, 1).start()
    idx_desc(0, 0).wait()
    for d in gather_descs(0, 0, 0):
        d.start()

    def body(t, carry):
        s0 = t * NIBUF
        for k in range(NIBUF):
            s = s0 + k
            idx_desc(s + 2, (k + 2) % NIBUF).start()
            idx_desc(s + 1, (k + 1) % NIBUF).wait()
            for d in gather_descs(s + 1, (k + 1) % NVBUF, (k + 1) % NIBUF):
                d.start()
            for d in gather_descs(s, k % NVBUF, k % NIBUF):
                d.wait()
            accum(k % NVBUF)
        return carry

    lax.fori_loop(0, (SEQ - 4) // NIBUF, body, 0)

    for k in range(NIBUF):
        s = SEQ - 4 + k
        if k + 2 < NIBUF:
            idx_desc(s + 2, (k + 2) % NIBUF).start()
        if k + 1 < NIBUF:
            idx_desc(s + 1, (k + 1) % NIBUF).wait()
            for d in gather_descs(s + 1, (k + 1) % NVBUF, (k + 1) % NIBUF):
                d.start()
        for d in gather_descs(s, k % NVBUF, k % NIBUF):
            d.wait()
        accum(k % NVBUF)

    pltpu.sync_copy(acc_v, out_hbm.at[pl.ds(wid * BPW, BPW)])


@jax.jit
def kernel(input, lut, bias):
    inp = input
    lut_flat = lut.reshape(1, VOCAB)
    run = pl.kernel(
        _lr_body,
        out_type=jax.ShapeDtypeStruct((BATCH,), jnp.float32),
        mesh=plsc.VectorSubcoreMesh(core_axis_name="c", subcore_axis_name="s"),
        scratch_types=[
            pltpu.VMEM_SHARED((VOCAB,), jnp.float32),
            pltpu.VMEM((NIBUF, BPW), jnp.int32),
            pltpu.VMEM((NVBUF, NCHUNK, 128), jnp.float32),
            pltpu.VMEM((BPW,), jnp.float32),
            pltpu.VMEM((LANES,), jnp.float32),
            pltpu.SemaphoreType.DMA,
            pltpu.SemaphoreType.DMA,
            pltpu.SemaphoreType.DMA,
            pltpu.SemaphoreType.DMA,
            pltpu.SemaphoreType.DMA,
            pltpu.SemaphoreType.DMA,
        ],
    )
    return run(inp, lut_flat, bias)


# speedup vs baseline: 1.6069x; 1.1309x over previous
import functools

import jax
import jax.numpy as jnp
from jax import lax
from jax.experimental import pallas as pl
from jax.experimental.pallas import tpu as pltpu
from jax.experimental.pallas import tpu_sc as plsc

SEQ = 200
BATCH = 16384
VOCAB = 1000000
NC = 2
NS = 16
NW = NC * NS
BPW = BATCH // NW
NCHUNK = BPW // 128
LANES = 16


NIBUF = 4
NVBUF = 2


def _lr_body(inp_hbm, lut_hbm, bias_hbm, out_hbm,
             table_sh, idx_v, vals_v, acc_v, bias_sm,
             isem0, isem1, isem2, isem3, gsem0, gsem1):
    cid = lax.axis_index("c")
    sid = lax.axis_index("s")
    wid = cid * NS + sid
    isems = (isem0, isem1, isem2, isem3)
    gsems = (gsem0, gsem1)

    @pl.when(sid == 0)
    def _():
        pltpu.sync_copy(lut_hbm.at[0], table_sh)

    bias_sm[...] = jnp.zeros((LANES,), jnp.float32)
    pltpu.sync_copy(bias_hbm, bias_sm.at[pl.ds(0, 1)])
    plsc.subcore_barrier()

    zero_idx = jnp.zeros((LANES,), jnp.int32)
    bvec = bias_sm[...].at[zero_idx].get(mode="promise_in_bounds")
    for i in range(BPW // LANES):
        acc_v[pl.ds(i * LANES, LANES)] = bvec

    def idx_desc(s, islot):
        return pltpu.make_async_copy(
            inp_hbm.at[s, pl.ds(wid * BPW, BPW)], idx_v.at[islot],
            isems[islot])

    def fire_gathers(vslot, islot):
        for j in range(NCHUNK):
            pltpu.async_copy(
                table_sh.at[idx_v.at[islot, pl.ds(j * 128, 128)]],
                acc_v.at[pl.ds(j * 128, 128)], gsems[vslot], add=True)

    def wait_gathers(vslot, islot):
        for j in range(NCHUNK):
            pltpu.make_async_copy(
                table_sh.at[idx_v.at[islot, pl.ds(j * 128, 128)]],
                acc_v.at[pl.ds(j * 128, 128)], gsems[vslot]).wait()


    idx_desc(0, 0).start()
    idx_desc(1, 1).start()
    idx_desc(0, 0).wait()
    fire_gathers(0, 0)

    def body(t, carry):
        s0 = t * NIBUF
        for k in range(NIBUF):
            s = s0 + k
            idx_desc(s + 2, (k + 2) % NIBUF).start()
            idx_desc(s + 1, (k + 1) % NIBUF).wait()
            fire_gathers((k + 1) % NVBUF, (k + 1) % NIBUF)
            wait_gathers(k % NVBUF, k % NIBUF)
        return carry

    lax.fori_loop(0, (SEQ - 4) // NIBUF, body, 0)

    for k in range(NIBUF):
        s = SEQ - 4 + k
        if k + 2 < NIBUF:
            idx_desc(s + 2, (k + 2) % NIBUF).start()
        if k + 1 < NIBUF:
            idx_desc(s + 1, (k + 1) % NIBUF).wait()
            fire_gathers((k + 1) % NVBUF, (k + 1) % NIBUF)
        wait_gathers(k % NVBUF, k % NIBUF)

    pltpu.sync_copy(acc_v, out_hbm.at[pl.ds(wid * BPW, BPW)])


@jax.jit
def kernel(input, lut, bias):
    inp = input
    lut_flat = lut.reshape(1, VOCAB)
    run = pl.kernel(
        _lr_body,
        out_type=jax.ShapeDtypeStruct((BATCH,), jnp.float32),
        mesh=plsc.VectorSubcoreMesh(core_axis_name="c", subcore_axis_name="s"),
        scratch_types=[
            pltpu.VMEM_SHARED((VOCAB,), jnp.float32),
            pltpu.VMEM((NIBUF, BPW), jnp.int32),
            pltpu.VMEM((NVBUF, NCHUNK, 128), jnp.float32),
            pltpu.VMEM((BPW,), jnp.float32),
            pltpu.VMEM((LANES,), jnp.float32),
            pltpu.SemaphoreType.DMA,
            pltpu.SemaphoreType.DMA,
            pltpu.SemaphoreType.DMA,
            pltpu.SemaphoreType.DMA,
            pltpu.SemaphoreType.DMA,
            pltpu.SemaphoreType.DMA,
        ],
    )
    return run(inp, lut_flat, bias)


# speedup vs baseline: 2.3974x; 1.4920x over previous
import functools

import jax
import jax.numpy as jnp
from jax import lax
from jax.experimental import pallas as pl
from jax.experimental.pallas import tpu as pltpu
from jax.experimental.pallas import tpu_sc as plsc

SEQ = 200
BATCH = 16384
VOCAB = 1000000
NC = 2
NS = 16
NW = NC * NS
BPW = BATCH // NW
NCHUNK = BPW // 128
LANES = 16
RCHUNK = 40
NRC = SEQ // RCHUNK


def _lr_body(inp_hbm, lut_hbm, bias_hbm, out_hbm,
             table_sh, idx_all, acc_v, bias_sm,
             csem0, csem1, csem2, csem3, csem4, gsem):
    cid = lax.axis_index("c")
    sid = lax.axis_index("s")
    wid = cid * NS + sid
    csems = (csem0, csem1, csem2, csem3, csem4)

    def chunk_desc(c, slot):
        return pltpu.make_async_copy(
            inp_hbm.at[pl.ds(c * RCHUNK, RCHUNK), pl.ds(wid * BPW, BPW)],
            idx_all.at[slot], csems[c])

    chunk_desc(0, 0).start()
    chunk_desc(1, 1).start()

    @pl.when(sid == 0)
    def _():
        pltpu.sync_copy(lut_hbm.at[0], table_sh)

    bias_sm[...] = jnp.zeros((LANES,), jnp.float32)
    pltpu.sync_copy(bias_hbm, bias_sm.at[pl.ds(0, 1)])

    zero_idx = jnp.zeros((LANES,), jnp.int32)
    bvec = bias_sm[...].at[zero_idx].get(mode="promise_in_bounds")
    for i in range(BPW // LANES):
        acc_v[pl.ds(i * LANES, LANES)] = bvec

    plsc.subcore_barrier()

    def fire_row(slot):
        def f(s, carry):
            for j in range(NCHUNK):
                pltpu.async_copy(
                    table_sh.at[idx_all.at[slot, s, pl.ds(j * 128, 128)]],
                    acc_v.at[pl.ds(j * 128, 128)], gsem, add=True)
            return carry
        return f

    def drain_row(slot):
        def f(s, carry):
            for j in range(NCHUNK):
                pltpu.make_async_copy(
                    table_sh.at[idx_all.at[slot, s, pl.ds(j * 128, 128)]],
                    acc_v.at[pl.ds(j * 128, 128)], gsem).wait()
            return carry
        return f

    for c in range(NRC):
        chunk_desc(c, c % 3).wait()
        lax.fori_loop(0, RCHUNK, fire_row(c % 3), 0)
        if c >= 1:
            lax.fori_loop(0, RCHUNK, drain_row((c - 1) % 3), 0)
        if c + 2 < NRC:
            chunk_desc(c + 2, (c + 2) % 3).start()

    lax.fori_loop(0, RCHUNK, drain_row((NRC - 1) % 3), 0)

    pltpu.sync_copy(acc_v, out_hbm.at[pl.ds(wid * BPW, BPW)])


@jax.jit
def kernel(input, lut, bias):
    lut_flat = lut.reshape(1, VOCAB)
    run = pl.kernel(
        _lr_body,
        out_type=jax.ShapeDtypeStruct((BATCH,), jnp.float32),
        mesh=plsc.VectorSubcoreMesh(core_axis_name="c", subcore_axis_name="s"),
        scratch_types=[
            pltpu.VMEM_SHARED((VOCAB,), jnp.float32),
            pltpu.VMEM((3, RCHUNK, BPW), jnp.int32),
            pltpu.VMEM((BPW,), jnp.float32),
            pltpu.VMEM((LANES,), jnp.float32),
            pltpu.SemaphoreType.DMA,
            pltpu.SemaphoreType.DMA,
            pltpu.SemaphoreType.DMA,
            pltpu.SemaphoreType.DMA,
            pltpu.SemaphoreType.DMA,
            pltpu.SemaphoreType.DMA,
        ],
    )
    return run(input, lut_flat, bias)
